# baseline (speedup 1.0000x reference)
import functools

import jax
import jax.numpy as jnp
from jax import lax
from jax.experimental import pallas as pl
from jax.experimental.pallas import tpu as pltpu
from jax.experimental.pallas import tpu_sc as plsc

VOCAB_SIZE = 100000
D_MODEL = 128
BATCH = 16384
SEQ = 50

NC = 2
NS = 16
NW = NC * NS

ROWS_W = BATCH // NW
NB = 8
NGRP = ROWS_W // NB


def _gather_body(toks_hbm, table_hbm, out_hbm, idx_v, rows_v, gsem, wsem):
    c = lax.axis_index("c")
    s = lax.axis_index("s")
    wid = s * NC + c
    row0 = wid * ROWS_W

    pltpu.sync_copy(toks_hbm.at[wid], idx_v)

    def wait_gather(b, j):
        pltpu.make_async_copy(
            table_hbm.at[idx_v.at[j]], rows_v.at[b], gsem.at[b]
        ).wait()

    def wait_write(b):
        pltpu.make_async_copy(rows_v.at[b], out_hbm.at[0], wsem.at[b]).wait()

    def start_gather(b, j):
        pltpu.async_copy(table_hbm.at[idx_v.at[j]], rows_v.at[b], gsem.at[b])

    def start_write(b, j):
        pltpu.async_copy(rows_v.at[b], out_hbm.at[row0 + j], wsem.at[b])

    for b in range(NB):
        start_gather(b, b)

    def group_step(g, carry):
        for b in range(NB):
            j = g * NB + b
            wait_gather(b, j)
            start_write(b, j)
        for b in range(NB):
            wait_write(b)
            start_gather(b, (g + 1) * NB + b)
        return carry

    lax.fori_loop(0, NGRP - 1, group_step, 0)

    for b in range(NB):
        j = (NGRP - 1) * NB + b
        wait_gather(b, j)
        start_write(b, j)
    for b in range(NB):
        wait_write(b)


@jax.jit
def _embed(toks_chunked, emb_weight):
    mesh = plsc.VectorSubcoreMesh(core_axis_name="c", subcore_axis_name="s")
    k = functools.partial(
        pl.kernel,
        out_type=jax.ShapeDtypeStruct((BATCH, SEQ, D_MODEL), jnp.float32),
        mesh=mesh,
        compiler_params=pltpu.CompilerParams(use_tc_tiling_on_sc=True),
        scratch_types=[
            pltpu.VMEM((ROWS_W, SEQ), jnp.int32),
            pltpu.VMEM((NB, SEQ, D_MODEL), jnp.float32),
            pltpu.SemaphoreType.DMA((NB,)),
            pltpu.SemaphoreType.DMA((NB,)),
        ],
    )(_gather_body)
    return k(toks_chunked, emb_weight)


def kernel(toks, emb_weight):
    toks_chunked = toks.astype(jnp.int32).reshape(NW, ROWS_W, SEQ)
    return _embed(toks_chunked, emb_weight)


# speedup vs baseline: 1.9294x; 1.9294x over previous
import functools

import jax
import jax.numpy as jnp
from jax import lax
from jax.experimental import pallas as pl
from jax.experimental.pallas import tpu as pltpu
from jax.experimental.pallas import tpu_sc as plsc

VOCAB_SIZE = 100000
D_MODEL = 128
BATCH = 16384
SEQ = 50

NC = 2
NS = 16
NW = NC * NS

COLS_W = BATCH // NW
CH = 128
S_BLK = COLS_W // CH
NCH = SEQ * S_BLK
NB = 4
NGRP = NCH // NB


def _gather_body(tokT_hbm, table_hbm, outP_hbm, idx_v, rows_v, gsem, wsem):
    c = lax.axis_index("c")
    s = lax.axis_index("s")
    wid = s * NC + c
    b0 = wid * COLS_W

    pltpu.sync_copy(tokT_hbm.at[:, pl.ds(b0, COLS_W)], idx_v)

    def wait_gather(bf):
        pltpu.make_async_copy(
            table_hbm.at[idx_v.at[0, pl.ds(0, CH)]], rows_v.at[bf], gsem.at[bf]
        ).wait()

    def wait_write(bf):
        pltpu.make_async_copy(
            rows_v.at[bf], outP_hbm.at[0, pl.ds(0, CH)], wsem.at[bf]
        ).wait()

    def start_gather(bf, j):
        sq = j // S_BLK
        bb = j % S_BLK
        idx = idx_v.at[sq, pl.ds(bb * CH, CH)]
        pltpu.async_copy(table_hbm.at[idx], rows_v.at[bf], gsem.at[bf])

    def start_write(bf, j):
        sq = j // S_BLK
        bb = j % S_BLK
        pltpu.async_copy(
            rows_v.at[bf], outP_hbm.at[sq, pl.ds(b0 + bb * CH, CH)], wsem.at[bf]
        )

    for bf in range(NB):
        start_gather(bf, bf)

    def group_step(g, carry):
        for bf in range(NB):
            j = g * NB + bf
            wait_gather(bf)
            start_write(bf, j)
        for bf in range(NB):
            wait_write(bf)
            start_gather(bf, (g + 1) * NB + bf)
        return carry

    lax.fori_loop(0, NGRP - 1, group_step, 0)

    for bf in range(NB):
        j = (NGRP - 1) * NB + bf
        wait_gather(bf)
        start_write(bf, j)
    for bf in range(NB):
        wait_write(bf)


@jax.jit
def _embed(tokT, emb_weight):
    mesh = plsc.VectorSubcoreMesh(core_axis_name="c", subcore_axis_name="s")
    k = functools.partial(
        pl.kernel,
        out_type=jax.ShapeDtypeStruct((SEQ, BATCH, D_MODEL), jnp.float32),
        mesh=mesh,
        scratch_types=[
            pltpu.VMEM((SEQ, COLS_W), jnp.int32),
            pltpu.VMEM((NB, CH, D_MODEL), jnp.float32),
            pltpu.SemaphoreType.DMA((NB,)),
            pltpu.SemaphoreType.DMA((NB,)),
        ],
    )(_gather_body)
    return k(tokT, emb_weight)


def kernel(toks, emb_weight):
    tokT = toks.astype(jnp.int32).T
    outP = _embed(tokT, emb_weight)
    return outP.transpose(1, 0, 2)
